# baseline (speedup 1.0000x reference)
import functools

import jax
import jax.numpy as jnp
from jax import lax
from jax.experimental import pallas as pl
from jax.experimental.pallas import tpu as pltpu
from jax.experimental.pallas import tpu_sc as plsc

_BS = 512


def _add_kernel(x_ref, pos_ref, out_ref):
    out_ref[...] = x_ref[...] + pos_ref[...]


def _kernel_tc(x, pos_table):
    batch, seq_len, d_model = x.shape
    bs = _BS
    num_blocks = seq_len // bs
    grid = (num_blocks, batch)
    return pl.pallas_call(
        _add_kernel,
        grid=grid,
        in_specs=[
            pl.BlockSpec((1, bs, d_model), lambda i, b: (b, i, 0)),
            pl.BlockSpec((bs, d_model), lambda i, b: (i, 0)),
        ],
        out_specs=pl.BlockSpec((1, bs, d_model), lambda i, b: (b, i, 0)),
        out_shape=jax.ShapeDtypeStruct((batch, seq_len, d_model), x.dtype),
        compiler_params=pltpu.CompilerParams(
            dimension_semantics=("parallel", "arbitrary"),
        ),
    )(x, pos_table)


_NC = 2
_NS = 16
_T = 8
_U = 16


def _sc_body(n_rows, seq_len, d_model, x_hbm, pos_hbm, out_hbm, xbuf, pbuf, semx, semp):
    nw = _NC * _NS
    rows_per_w = n_rows // nw
    wid = lax.axis_index("s") * _NC + lax.axis_index("c")
    base_row = wid * rows_per_w
    pos_base_row = lax.rem(base_row, seq_len)
    chunk_elems = _T * d_model
    nchunks = rows_per_w // _T

    def chunk(i, carry):
        rb = (base_row + i * _T) * d_model
        pb = (pos_base_row + i * _T) * d_model
        cx = pltpu.async_copy(x_hbm.at[pl.ds(rb, chunk_elems)], xbuf, semx)
        cp = pltpu.async_copy(pos_hbm.at[pl.ds(pb, chunk_elems)], pbuf, semp)
        cx.wait()
        cp.wait()

        def slices(j, c):
            for u in range(_U):
                off = (j * _U + u) * 16
                v = pbuf[pl.ds(off, 16)]
                plsc.addupdate(xbuf.at[pl.ds(off, 16)], v)
            return c

        lax.fori_loop(0, chunk_elems // 16 // _U, slices, 0)
        pltpu.sync_copy(xbuf, out_hbm.at[pl.ds(rb, chunk_elems)])
        return carry

    lax.fori_loop(0, nchunks, chunk, 0)


def _kernel_sc(x, pos_table):
    batch, seq_len, d_model = x.shape
    n_rows = batch * seq_len
    x_flat = x.reshape(n_rows * d_model)
    pos_flat = pos_table.reshape(-1)
    mesh = plsc.VectorSubcoreMesh(core_axis_name="c", subcore_axis_name="s")
    chunk_elems = _T * d_model
    run = pl.kernel(
        functools.partial(_sc_body, n_rows, seq_len, d_model),
        out_type=jax.ShapeDtypeStruct((n_rows * d_model,), x.dtype),
        mesh=mesh,
        scratch_types=[
            pltpu.VMEM((chunk_elems,), jnp.float32),
            pltpu.VMEM((chunk_elems,), jnp.float32),
            pltpu.SemaphoreType.DMA,
            pltpu.SemaphoreType.DMA,
        ],
    )
    out = run(x_flat, pos_flat)
    return out.reshape(batch, seq_len, d_model)


def kernel(x, pos_table):
    return _kernel_sc(x, pos_table)


# speedup vs baseline: 1.3642x; 1.3642x over previous
import functools

import jax
import jax.numpy as jnp
from jax import lax
from jax.experimental import pallas as pl
from jax.experimental.pallas import tpu as pltpu
from jax.experimental.pallas import tpu_sc as plsc

_BS = 512


def _add_kernel(x_ref, pos_ref, out_ref):
    out_ref[...] = x_ref[...] + pos_ref[...]


def _kernel_tc(x, pos_table):
    batch, seq_len, d_model = x.shape
    bs = _BS
    num_blocks = seq_len // bs
    grid = (num_blocks, batch)
    return pl.pallas_call(
        _add_kernel,
        grid=grid,
        in_specs=[
            pl.BlockSpec((1, bs, d_model), lambda i, b: (b, i, 0)),
            pl.BlockSpec((bs, d_model), lambda i, b: (i, 0)),
        ],
        out_specs=pl.BlockSpec((1, bs, d_model), lambda i, b: (b, i, 0)),
        out_shape=jax.ShapeDtypeStruct((batch, seq_len, d_model), x.dtype),
        compiler_params=pltpu.CompilerParams(
            dimension_semantics=("parallel", "arbitrary"),
        ),
    )(x, pos_table)


_NC = 2
_NS = 16
_T = 8
_U = 16


def _sc_body(n_rows, seq_len, d_model, x_hbm, pos_hbm, out_hbm, xbuf, pbuf, semx, semp):
    nw = _NC * _NS
    rows_per_w = n_rows // nw
    wid = lax.axis_index("s") * _NC + lax.axis_index("c")
    base_row = wid * rows_per_w
    pos_base_row = lax.rem(base_row, seq_len)
    chunk_elems = _T * d_model
    nchunks = rows_per_w // _T

    def chunk(i, carry):
        rb = (base_row + i * _T) * d_model
        pb = (pos_base_row + i * _T) * d_model
        cx = pltpu.async_copy(x_hbm.at[pl.ds(rb, chunk_elems)], xbuf, semx)
        cp = pltpu.async_copy(pos_hbm.at[pl.ds(pb, chunk_elems)], pbuf, semp)
        cx.wait()
        cp.wait()

        def slices(j, c):
            for u in range(_U):
                off = (j * _U + u) * 16
                v = pbuf[pl.ds(off, 16)]
                plsc.addupdate(xbuf.at[pl.ds(off, 16)], v)
            return c

        lax.fori_loop(0, chunk_elems // 16 // _U, slices, 0)
        pltpu.sync_copy(xbuf, out_hbm.at[pl.ds(rb, chunk_elems)])
        return carry

    lax.fori_loop(0, nchunks, chunk, 0)


def _kernel_sc(x, pos_table):
    batch, seq_len, d_model = x.shape
    n_rows = batch * seq_len
    x_flat = x.reshape(n_rows * d_model)
    pos_flat = pos_table.reshape(-1)
    mesh = plsc.VectorSubcoreMesh(core_axis_name="c", subcore_axis_name="s")
    chunk_elems = _T * d_model
    run = pl.kernel(
        functools.partial(_sc_body, n_rows, seq_len, d_model),
        out_type=jax.ShapeDtypeStruct((n_rows * d_model,), x.dtype),
        mesh=mesh,
        scratch_types=[
            pltpu.VMEM((chunk_elems,), jnp.float32),
            pltpu.VMEM((chunk_elems,), jnp.float32),
            pltpu.SemaphoreType.DMA,
            pltpu.SemaphoreType.DMA,
        ],
    )
    out = run(x_flat, pos_flat)
    return out.reshape(batch, seq_len, d_model)


def _sc_body_slice(row0, n_rows, seq_len, d_model, x_hbm, pos_hbm, out_hbm,
                   xbuf, pbuf, semx, semp):
    nw = _NC * _NS
    rows_per_w = n_rows // nw
    wid = lax.axis_index("s") * _NC + lax.axis_index("c")
    base_row = wid * rows_per_w
    pos_base_row = lax.rem(row0 + base_row, seq_len)
    chunk_elems = _T * d_model
    nchunks = rows_per_w // _T

    def chunk(i, carry):
        rb = (base_row + i * _T) * d_model
        xb = (row0 + base_row + i * _T) * d_model
        pb = (pos_base_row + i * _T) * d_model
        cx = pltpu.async_copy(x_hbm.at[pl.ds(xb, chunk_elems)], xbuf, semx)
        cp = pltpu.async_copy(pos_hbm.at[pl.ds(pb, chunk_elems)], pbuf, semp)
        cx.wait()
        cp.wait()

        def slices(j, c):
            for u in range(_U):
                off = (j * _U + u) * 16
                v = pbuf[pl.ds(off, 16)]
                plsc.addupdate(xbuf.at[pl.ds(off, 16)], v)
            return c

        lax.fori_loop(0, chunk_elems // 16 // _U, slices, 0)
        pltpu.sync_copy(xbuf, out_hbm.at[pl.ds(rb, chunk_elems)])
        return carry

    lax.fori_loop(0, nchunks, chunk, 0)


def _kernel_hybrid(x, pos_table, sc_batches=1):
    batch, seq_len, d_model = x.shape
    tc_batches = batch - sc_batches
    x_flat = x.reshape(-1)
    pos_flat = pos_table.reshape(-1)

    bs = _BS
    out_tc = pl.pallas_call(
        _add_kernel,
        grid=(seq_len // bs, tc_batches),
        in_specs=[
            pl.BlockSpec((1, bs, d_model), lambda i, b: (b, i, 0)),
            pl.BlockSpec((bs, d_model), lambda i, b: (i, 0)),
        ],
        out_specs=pl.BlockSpec((1, bs, d_model), lambda i, b: (b, i, 0)),
        out_shape=jax.ShapeDtypeStruct((tc_batches, seq_len, d_model), x.dtype),
        compiler_params=pltpu.CompilerParams(
            dimension_semantics=("parallel", "arbitrary"),
        ),
    )(x, pos_table)

    n_rows = sc_batches * seq_len
    row0 = tc_batches * seq_len
    mesh = plsc.VectorSubcoreMesh(core_axis_name="c", subcore_axis_name="s")
    chunk_elems = _T * d_model
    run = pl.kernel(
        functools.partial(_sc_body_slice, row0, n_rows, seq_len, d_model),
        out_type=jax.ShapeDtypeStruct((n_rows * d_model,), x.dtype),
        mesh=mesh,
        scratch_types=[
            pltpu.VMEM((chunk_elems,), jnp.float32),
            pltpu.VMEM((chunk_elems,), jnp.float32),
            pltpu.SemaphoreType.DMA,
            pltpu.SemaphoreType.DMA,
        ],
    )
    out_sc = run(x_flat, pos_flat).reshape(sc_batches, seq_len, d_model)
    return jnp.concatenate([out_tc, out_sc], axis=0)


def kernel(x, pos_table):
    return _kernel_hybrid(x, pos_table)


# speedup vs baseline: 5.2802x; 3.8706x over previous
import functools

import jax
import jax.numpy as jnp
from jax import lax
from jax.experimental import pallas as pl
from jax.experimental.pallas import tpu as pltpu
from jax.experimental.pallas import tpu_sc as plsc

_BS = 512


def _add_kernel(x_ref, pos_ref, out_ref):
    out_ref[...] = x_ref[...] + pos_ref[...]


def _kernel_tc(x, pos_table):
    batch, seq_len, d_model = x.shape
    bs = _BS
    num_blocks = seq_len // bs
    grid = (num_blocks, batch)
    return pl.pallas_call(
        _add_kernel,
        grid=grid,
        in_specs=[
            pl.BlockSpec((1, bs, d_model), lambda i, b: (b, i, 0)),
            pl.BlockSpec((bs, d_model), lambda i, b: (i, 0)),
        ],
        out_specs=pl.BlockSpec((1, bs, d_model), lambda i, b: (b, i, 0)),
        out_shape=jax.ShapeDtypeStruct((batch, seq_len, d_model), x.dtype),
        compiler_params=pltpu.CompilerParams(
            dimension_semantics=("parallel", "arbitrary"),
            vmem_limit_bytes=128 * 1024 * 1024,
        ),
    )(x, pos_table)


_NC = 2
_NS = 16
_T = 8
_U = 16


def _sc_body(n_rows, seq_len, d_model, x_hbm, pos_hbm, out_hbm, xbuf, pbuf, semx, semp):
    nw = _NC * _NS
    rows_per_w = n_rows // nw
    wid = lax.axis_index("s") * _NC + lax.axis_index("c")
    base_row = wid * rows_per_w
    pos_base_row = lax.rem(base_row, seq_len)
    chunk_elems = _T * d_model
    nchunks = rows_per_w // _T

    def chunk(i, carry):
        rb = (base_row + i * _T) * d_model
        pb = (pos_base_row + i * _T) * d_model
        cx = pltpu.async_copy(x_hbm.at[pl.ds(rb, chunk_elems)], xbuf, semx)
        cp = pltpu.async_copy(pos_hbm.at[pl.ds(pb, chunk_elems)], pbuf, semp)
        cx.wait()
        cp.wait()

        def slices(j, c):
            for u in range(_U):
                off = (j * _U + u) * 16
                v = pbuf[pl.ds(off, 16)]
                plsc.addupdate(xbuf.at[pl.ds(off, 16)], v)
            return c

        lax.fori_loop(0, chunk_elems // 16 // _U, slices, 0)
        pltpu.sync_copy(xbuf, out_hbm.at[pl.ds(rb, chunk_elems)])
        return carry

    lax.fori_loop(0, nchunks, chunk, 0)


def _kernel_sc(x, pos_table):
    batch, seq_len, d_model = x.shape
    n_rows = batch * seq_len
    x_flat = x.reshape(n_rows * d_model)
    pos_flat = pos_table.reshape(-1)
    mesh = plsc.VectorSubcoreMesh(core_axis_name="c", subcore_axis_name="s")
    chunk_elems = _T * d_model
    run = pl.kernel(
        functools.partial(_sc_body, n_rows, seq_len, d_model),
        out_type=jax.ShapeDtypeStruct((n_rows * d_model,), x.dtype),
        mesh=mesh,
        scratch_types=[
            pltpu.VMEM((chunk_elems,), jnp.float32),
            pltpu.VMEM((chunk_elems,), jnp.float32),
            pltpu.SemaphoreType.DMA,
            pltpu.SemaphoreType.DMA,
        ],
    )
    out = run(x_flat, pos_flat)
    return out.reshape(batch, seq_len, d_model)


def _sc_body_slice(row0, n_rows, seq_len, d_model, x_hbm, pos_hbm, out_hbm,
                   xbuf, pbuf, semx, semp):
    nw = _NC * _NS
    rows_per_w = n_rows // nw
    wid = lax.axis_index("s") * _NC + lax.axis_index("c")
    base_row = wid * rows_per_w
    pos_base_row = lax.rem(row0 + base_row, seq_len)
    chunk_elems = _T * d_model
    nchunks = rows_per_w // _T

    def chunk(i, carry):
        rb = (base_row + i * _T) * d_model
        xb = (row0 + base_row + i * _T) * d_model
        pb = (pos_base_row + i * _T) * d_model
        cx = pltpu.async_copy(x_hbm.at[pl.ds(xb, chunk_elems)], xbuf, semx)
        cp = pltpu.async_copy(pos_hbm.at[pl.ds(pb, chunk_elems)], pbuf, semp)
        cx.wait()
        cp.wait()

        def slices(j, c):
            for u in range(_U):
                off = (j * _U + u) * 16
                v = pbuf[pl.ds(off, 16)]
                plsc.addupdate(xbuf.at[pl.ds(off, 16)], v)
            return c

        lax.fori_loop(0, chunk_elems // 16 // _U, slices, 0)
        pltpu.sync_copy(xbuf, out_hbm.at[pl.ds(rb, chunk_elems)])
        return carry

    lax.fori_loop(0, nchunks, chunk, 0)


def _kernel_hybrid(x, pos_table, sc_batches=1):
    batch, seq_len, d_model = x.shape
    tc_batches = batch - sc_batches
    x_flat = x.reshape(-1)
    pos_flat = pos_table.reshape(-1)

    bs = _BS
    out_tc = pl.pallas_call(
        _add_kernel,
        grid=(seq_len // bs, tc_batches),
        in_specs=[
            pl.BlockSpec((1, bs, d_model), lambda i, b: (b, i, 0)),
            pl.BlockSpec((bs, d_model), lambda i, b: (i, 0)),
        ],
        out_specs=pl.BlockSpec((1, bs, d_model), lambda i, b: (b, i, 0)),
        out_shape=jax.ShapeDtypeStruct((tc_batches, seq_len, d_model), x.dtype),
        compiler_params=pltpu.CompilerParams(
            dimension_semantics=("parallel", "arbitrary"),
            vmem_limit_bytes=128 * 1024 * 1024,
        ),
    )(x, pos_table)

    n_rows = sc_batches * seq_len
    row0 = tc_batches * seq_len
    mesh = plsc.VectorSubcoreMesh(core_axis_name="c", subcore_axis_name="s")
    chunk_elems = _T * d_model
    run = pl.kernel(
        functools.partial(_sc_body_slice, row0, n_rows, seq_len, d_model),
        out_type=jax.ShapeDtypeStruct((n_rows * d_model,), x.dtype),
        mesh=mesh,
        scratch_types=[
            pltpu.VMEM((chunk_elems,), jnp.float32),
            pltpu.VMEM((chunk_elems,), jnp.float32),
            pltpu.SemaphoreType.DMA,
            pltpu.SemaphoreType.DMA,
        ],
    )
    out_sc = run(x_flat, pos_flat).reshape(sc_batches, seq_len, d_model)
    return jnp.concatenate([out_tc, out_sc], axis=0)


_NBUF = 6
_RB = 256


def _x_copy(x, i, batch, xbuf, xsems):
    j, b = i // batch, i % batch
    return pltpu.make_async_copy(
        x.at[b, pl.ds(j * _RB, _RB), :], xbuf.at[i % _NBUF], xsems.at[i % _NBUF]
    )


def _out_copy(out, i, batch, xbuf, osems):
    j, b = i // batch, i % batch
    return pltpu.make_async_copy(
        xbuf.at[i % _NBUF], out.at[b, pl.ds(j * _RB, _RB), :], osems.at[i % _NBUF]
    )


def _pos_copy(pos, j, pbuf, psems):
    return pltpu.make_async_copy(
        pos.at[pl.ds(j * _RB, _RB), :], pbuf.at[j % 2], psems.at[j % 2]
    )


def _manual_body(x_hbm, pos_hbm, out_hbm, xbuf, pbuf, xsems, psems, osems):
    nj = pl.num_programs(0)
    batch = pl.num_programs(1)
    n = nj * batch
    j = pl.program_id(0)
    b = pl.program_id(1)
    i = j * batch + b
    look = _NBUF - 1

    @pl.when(i == 0)
    def _prime():
        for p in range(look):
            _x_copy(x_hbm, p, batch, xbuf, xsems).start()
        _pos_copy(pos_hbm, 0, pbuf, psems).start()
        _pos_copy(pos_hbm, 1, pbuf, psems).start()

    ii = i + look

    @pl.when(jnp.logical_and(ii < n, ii >= _NBUF))
    def _drain_slot():
        _out_copy(out_hbm, ii - _NBUF, batch, xbuf, osems).wait()

    @pl.when(ii < n)
    def _issue_in():
        _x_copy(x_hbm, ii, batch, xbuf, xsems).start()

    _x_copy(x_hbm, i, batch, xbuf, xsems).wait()

    @pl.when(b == 0)
    def _wait_pos():
        _pos_copy(pos_hbm, j, pbuf, psems).wait()

    k = i % _NBUF
    xbuf[k] = xbuf[k] + pbuf[j % 2]
    _out_copy(out_hbm, i, batch, xbuf, osems).start()

    @pl.when(jnp.logical_and(b == batch - 1, j + 2 < nj))
    def _prefetch_pos():
        _pos_copy(pos_hbm, j + 2, pbuf, psems).start()

    @pl.when(i == n - 1)
    def _drain_all():
        for p in range(_NBUF):
            _out_copy(out_hbm, n - _NBUF + p, batch, xbuf, osems).wait()


def _kernel_tc_manual(x, pos_table):
    batch, seq_len, d_model = x.shape
    nj = seq_len // _RB
    return pl.pallas_call(
        _manual_body,
        grid=(nj, batch),
        in_specs=[
            pl.BlockSpec(memory_space=pltpu.HBM),
            pl.BlockSpec(memory_space=pltpu.HBM),
        ],
        out_specs=pl.BlockSpec(memory_space=pltpu.HBM),
        out_shape=jax.ShapeDtypeStruct((batch, seq_len, d_model), x.dtype),
        scratch_shapes=[
            pltpu.VMEM((_NBUF, _RB, d_model), jnp.float32),
            pltpu.VMEM((2, _RB, d_model), jnp.float32),
            pltpu.SemaphoreType.DMA((_NBUF,)),
            pltpu.SemaphoreType.DMA((2,)),
            pltpu.SemaphoreType.DMA((_NBUF,)),
        ],
        compiler_params=pltpu.CompilerParams(
            dimension_semantics=("arbitrary", "arbitrary"),
        ),
    )(x, pos_table)


def kernel(x, pos_table):
    return _kernel_tc_manual(x, pos_table)


# speedup vs baseline: 5.5066x; 1.0429x over previous
import functools

import jax
import jax.numpy as jnp
from jax import lax
from jax.experimental import pallas as pl
from jax.experimental.pallas import tpu as pltpu
from jax.experimental.pallas import tpu_sc as plsc

_BS = 512


def _add_kernel(x_ref, pos_ref, out_ref):
    out_ref[...] = x_ref[...] + pos_ref[...]


def _kernel_tc(x, pos_table):
    batch, seq_len, d_model = x.shape
    bs = _BS
    num_blocks = seq_len // bs
    grid = (num_blocks, batch)
    return pl.pallas_call(
        _add_kernel,
        grid=grid,
        in_specs=[
            pl.BlockSpec((1, bs, d_model), lambda i, b: (b, i, 0)),
            pl.BlockSpec((bs, d_model), lambda i, b: (i, 0)),
        ],
        out_specs=pl.BlockSpec((1, bs, d_model), lambda i, b: (b, i, 0)),
        out_shape=jax.ShapeDtypeStruct((batch, seq_len, d_model), x.dtype),
        compiler_params=pltpu.CompilerParams(
            dimension_semantics=("parallel", "arbitrary"),
            vmem_limit_bytes=128 * 1024 * 1024,
        ),
    )(x, pos_table)


_NC = 2
_NS = 16
_T = 8
_U = 16


def _sc_body(n_rows, seq_len, d_model, x_hbm, pos_hbm, out_hbm, xbuf, pbuf, semx, semp):
    nw = _NC * _NS
    rows_per_w = n_rows // nw
    wid = lax.axis_index("s") * _NC + lax.axis_index("c")
    base_row = wid * rows_per_w
    pos_base_row = lax.rem(base_row, seq_len)
    chunk_elems = _T * d_model
    nchunks = rows_per_w // _T

    def chunk(i, carry):
        rb = (base_row + i * _T) * d_model
        pb = (pos_base_row + i * _T) * d_model
        cx = pltpu.async_copy(x_hbm.at[pl.ds(rb, chunk_elems)], xbuf, semx)
        cp = pltpu.async_copy(pos_hbm.at[pl.ds(pb, chunk_elems)], pbuf, semp)
        cx.wait()
        cp.wait()

        def slices(j, c):
            for u in range(_U):
                off = (j * _U + u) * 16
                v = pbuf[pl.ds(off, 16)]
                plsc.addupdate(xbuf.at[pl.ds(off, 16)], v)
            return c

        lax.fori_loop(0, chunk_elems // 16 // _U, slices, 0)
        pltpu.sync_copy(xbuf, out_hbm.at[pl.ds(rb, chunk_elems)])
        return carry

    lax.fori_loop(0, nchunks, chunk, 0)


def _kernel_sc(x, pos_table):
    batch, seq_len, d_model = x.shape
    n_rows = batch * seq_len
    x_flat = x.reshape(n_rows * d_model)
    pos_flat = pos_table.reshape(-1)
    mesh = plsc.VectorSubcoreMesh(core_axis_name="c", subcore_axis_name="s")
    chunk_elems = _T * d_model
    run = pl.kernel(
        functools.partial(_sc_body, n_rows, seq_len, d_model),
        out_type=jax.ShapeDtypeStruct((n_rows * d_model,), x.dtype),
        mesh=mesh,
        scratch_types=[
            pltpu.VMEM((chunk_elems,), jnp.float32),
            pltpu.VMEM((chunk_elems,), jnp.float32),
            pltpu.SemaphoreType.DMA,
            pltpu.SemaphoreType.DMA,
        ],
    )
    out = run(x_flat, pos_flat)
    return out.reshape(batch, seq_len, d_model)


def _sc_body_slice(row0, n_rows, seq_len, d_model, x_hbm, pos_hbm, out_hbm,
                   xbuf, pbuf, semx, semp):
    nw = _NC * _NS
    rows_per_w = n_rows // nw
    wid = lax.axis_index("s") * _NC + lax.axis_index("c")
    base_row = wid * rows_per_w
    pos_base_row = lax.rem(row0 + base_row, seq_len)
    chunk_elems = _T * d_model
    nchunks = rows_per_w // _T

    def chunk(i, carry):
        rb = (base_row + i * _T) * d_model
        xb = (row0 + base_row + i * _T) * d_model
        pb = (pos_base_row + i * _T) * d_model
        cx = pltpu.async_copy(x_hbm.at[pl.ds(xb, chunk_elems)], xbuf, semx)
        cp = pltpu.async_copy(pos_hbm.at[pl.ds(pb, chunk_elems)], pbuf, semp)
        cx.wait()
        cp.wait()

        def slices(j, c):
            for u in range(_U):
                off = (j * _U + u) * 16
                v = pbuf[pl.ds(off, 16)]
                plsc.addupdate(xbuf.at[pl.ds(off, 16)], v)
            return c

        lax.fori_loop(0, chunk_elems // 16 // _U, slices, 0)
        pltpu.sync_copy(xbuf, out_hbm.at[pl.ds(rb, chunk_elems)])
        return carry

    lax.fori_loop(0, nchunks, chunk, 0)


def _kernel_hybrid(x, pos_table, sc_batches=1):
    batch, seq_len, d_model = x.shape
    tc_batches = batch - sc_batches
    x_flat = x.reshape(-1)
    pos_flat = pos_table.reshape(-1)

    bs = _BS
    out_tc = pl.pallas_call(
        _add_kernel,
        grid=(seq_len // bs, tc_batches),
        in_specs=[
            pl.BlockSpec((1, bs, d_model), lambda i, b: (b, i, 0)),
            pl.BlockSpec((bs, d_model), lambda i, b: (i, 0)),
        ],
        out_specs=pl.BlockSpec((1, bs, d_model), lambda i, b: (b, i, 0)),
        out_shape=jax.ShapeDtypeStruct((tc_batches, seq_len, d_model), x.dtype),
        compiler_params=pltpu.CompilerParams(
            dimension_semantics=("parallel", "arbitrary"),
            vmem_limit_bytes=128 * 1024 * 1024,
        ),
    )(x, pos_table)

    n_rows = sc_batches * seq_len
    row0 = tc_batches * seq_len
    mesh = plsc.VectorSubcoreMesh(core_axis_name="c", subcore_axis_name="s")
    chunk_elems = _T * d_model
    run = pl.kernel(
        functools.partial(_sc_body_slice, row0, n_rows, seq_len, d_model),
        out_type=jax.ShapeDtypeStruct((n_rows * d_model,), x.dtype),
        mesh=mesh,
        scratch_types=[
            pltpu.VMEM((chunk_elems,), jnp.float32),
            pltpu.VMEM((chunk_elems,), jnp.float32),
            pltpu.SemaphoreType.DMA,
            pltpu.SemaphoreType.DMA,
        ],
    )
    out_sc = run(x_flat, pos_flat).reshape(sc_batches, seq_len, d_model)
    return jnp.concatenate([out_tc, out_sc], axis=0)


_NBUF = 4
_RB = 512


def _x_copy(x, i, batch, xbuf, xsems):
    j, b = i // batch, i % batch
    return pltpu.make_async_copy(
        x.at[b, pl.ds(j * _RB, _RB), :], xbuf.at[i % _NBUF], xsems.at[i % _NBUF]
    )


def _out_copy(out, i, batch, xbuf, osems):
    j, b = i // batch, i % batch
    return pltpu.make_async_copy(
        xbuf.at[i % _NBUF], out.at[b, pl.ds(j * _RB, _RB), :], osems.at[i % _NBUF]
    )


def _pos_copy(pos, j, pbuf, psems):
    return pltpu.make_async_copy(
        pos.at[pl.ds(j * _RB, _RB), :], pbuf.at[j % 2], psems.at[j % 2]
    )


def _manual_body(x_hbm, pos_hbm, out_hbm, xbuf, pbuf, xsems, psems, osems):
    nj = pl.num_programs(0)
    batch = pl.num_programs(1)
    n = nj * batch
    j = pl.program_id(0)
    b = pl.program_id(1)
    i = j * batch + b
    look = _NBUF - 1

    @pl.when(i == 0)
    def _prime():
        for p in range(look):
            _x_copy(x_hbm, p, batch, xbuf, xsems).start()
        _pos_copy(pos_hbm, 0, pbuf, psems).start()
        _pos_copy(pos_hbm, 1, pbuf, psems).start()

    ii = i + look

    @pl.when(jnp.logical_and(ii < n, ii >= _NBUF))
    def _drain_slot():
        _out_copy(out_hbm, ii - _NBUF, batch, xbuf, osems).wait()

    @pl.when(ii < n)
    def _issue_in():
        _x_copy(x_hbm, ii, batch, xbuf, xsems).start()

    _x_copy(x_hbm, i, batch, xbuf, xsems).wait()

    @pl.when(b == 0)
    def _wait_pos():
        _pos_copy(pos_hbm, j, pbuf, psems).wait()

    k = i % _NBUF
    xbuf[k] = xbuf[k] + pbuf[j % 2]
    _out_copy(out_hbm, i, batch, xbuf, osems).start()

    @pl.when(jnp.logical_and(b == batch - 1, j + 2 < nj))
    def _prefetch_pos():
        _pos_copy(pos_hbm, j + 2, pbuf, psems).start()

    @pl.when(i == n - 1)
    def _drain_all():
        for p in range(_NBUF):
            _out_copy(out_hbm, n - _NBUF + p, batch, xbuf, osems).wait()


def _kernel_tc_manual(x, pos_table):
    batch, seq_len, d_model = x.shape
    nj = seq_len // _RB
    return pl.pallas_call(
        _manual_body,
        grid=(nj, batch),
        in_specs=[
            pl.BlockSpec(memory_space=pltpu.HBM),
            pl.BlockSpec(memory_space=pltpu.HBM),
        ],
        out_specs=pl.BlockSpec(memory_space=pltpu.HBM),
        out_shape=jax.ShapeDtypeStruct((batch, seq_len, d_model), x.dtype),
        scratch_shapes=[
            pltpu.VMEM((_NBUF, _RB, d_model), jnp.float32),
            pltpu.VMEM((2, _RB, d_model), jnp.float32),
            pltpu.SemaphoreType.DMA((_NBUF,)),
            pltpu.SemaphoreType.DMA((2,)),
            pltpu.SemaphoreType.DMA((_NBUF,)),
        ],
        compiler_params=pltpu.CompilerParams(
            dimension_semantics=("arbitrary", "arbitrary"),
        ),
    )(x, pos_table)


def kernel(x, pos_table):
    return _kernel_tc_manual(x, pos_table)
